# baseline (speedup 1.0000x reference)
import jax
import jax.numpy as jnp
from jax import lax
from jax.experimental import pallas as pl
from jax.experimental.pallas import tpu as pltpu
from jax.experimental.pallas import tpu_sc as plsc

B = 16384
F = 26
V = 100000
D = 32
NUM = 13
ED = F * D
FP = 28
NT = FP // 4
EPS = 1e-5

NC = 2
NS = 16
NW = NC * NS
ROWS_W = B // NW
RB = 128
NRB = ROWS_W // RB
CHUNKS_W = NRB * NT * 4
NCHUNK = (B // RB) * NT * 4
NBUF = 4


def _sc_gather_body(tab, idx2, out3, idx_v, rows_v, gsem):
    wid = lax.axis_index("s") * NC + lax.axis_index("c")
    pltpu.sync_copy(idx2.at[pl.ds(wid * CHUNKS_W, CHUNKS_W)], idx_v)
    b_base = wid * ROWS_W

    def outer(co, carry):
        c0 = co * NBUF
        for b in range(NBUF):
            pltpu.async_copy(tab.at[idx_v.at[c0 + b]], rows_v.at[b], gsem)
        for b in range(NBUF):
            c = c0 + b
            pltpu.make_async_copy(tab.at[idx_v.at[c]], rows_v.at[b], gsem).wait()
            rb = c // (NT * 4)
            ct = (c % (NT * 4)) // 4
            k = c % 4
            row = b_base + rb * RB
            pltpu.sync_copy(rows_v.at[b],
                            out3.at[ct, pl.ds(row, RB), pl.ds(32 * k, 32)])
        return carry

    lax.fori_loop(0, CHUNKS_W // NBUF, outer, 0)


_SC_GATHER_CACHE = []


def _sc_gather(tab_flat, flat_idx):
    if not _SC_GATHER_CACHE:
        _SC_GATHER_CACHE.append(pl.kernel(
            _sc_gather_body,
            out_type=jax.ShapeDtypeStruct((NT, B, 128), jnp.float32),
            mesh=plsc.VectorSubcoreMesh(core_axis_name="c", subcore_axis_name="s"),
            scratch_types=[
                pltpu.VMEM((CHUNKS_W, 128), jnp.int32),
                pltpu.VMEM((NBUF, 128, D), jnp.float32),
                pltpu.SemaphoreType.DMA,
            ],
            compiler_params=pltpu.CompilerParams(use_tc_tiling_on_sc=False),
        ))
    return _SC_GATHER_CACHE[0](tab_flat, flat_idx)


BB = 1024
_INV_STD = (1.0 + EPS) ** -0.5


def _mlp_body(x3, xn, w13, w1n, b1, g1, be1, w2, b2, g2, be2, w3, b3, out):
    h = jnp.dot(x3[0], w13[0], preferred_element_type=jnp.float32)
    for t in range(1, NT):
        h = h + jnp.dot(x3[t], w13[t], preferred_element_type=jnp.float32)
    h = h + jnp.dot(xn[...], w1n[...], preferred_element_type=jnp.float32)
    h = jnp.maximum(h + b1[...], 0.0)
    h = h * (g1[...] * _INV_STD) + be1[...]
    h = jnp.maximum(jnp.dot(h, w2[...], preferred_element_type=jnp.float32) + b2[...], 0.0)
    h = h * (g2[...] * _INV_STD) + be2[...]
    out[...] = jnp.dot(h, w3[...], preferred_element_type=jnp.float32) + b3[...]


_mlp = pl.pallas_call(
    _mlp_body,
    grid=(B // BB,),
    in_specs=[
        pl.BlockSpec((NT, BB, 128), lambda i: (0, i, 0)),
        pl.BlockSpec((BB, NUM), lambda i: (i, 0)),
        pl.BlockSpec((NT, 128, 128), lambda i: (0, 0, 0)),
        pl.BlockSpec((NUM, 128), lambda i: (0, 0)),
        pl.BlockSpec((1, 128), lambda i: (0, 0)),
        pl.BlockSpec((1, 128), lambda i: (0, 0)),
        pl.BlockSpec((1, 128), lambda i: (0, 0)),
        pl.BlockSpec((128, 64), lambda i: (0, 0)),
        pl.BlockSpec((1, 64), lambda i: (0, 0)),
        pl.BlockSpec((1, 64), lambda i: (0, 0)),
        pl.BlockSpec((1, 64), lambda i: (0, 0)),
        pl.BlockSpec((64, 1), lambda i: (0, 0)),
        pl.BlockSpec((1, 1), lambda i: (0, 0)),
    ],
    out_specs=pl.BlockSpec((BB, 1), lambda i: (i, 0)),
    out_shape=jax.ShapeDtypeStruct((B, 1), jnp.float32),
)


def kernel(x_cat, x_num, tables, W1, b1, g1, be1, W2, b2, g2, be2, W3, b3):
    offs = jnp.concatenate([jnp.arange(F, dtype=jnp.int32) * V,
                            jnp.zeros((FP - F,), jnp.int32)])
    xc = jnp.pad(x_cat, ((0, 0), (0, FP - F)))
    flat = xc + offs[None, :]
    flat_idx = (flat.reshape(B // RB, RB, NT, 4)
                    .transpose(0, 2, 3, 1)
                    .reshape(NCHUNK, 128))
    tab_flat = tables.reshape(F * V, D)
    x3 = _sc_gather(tab_flat, flat_idx)

    w1p = jnp.concatenate([W1[:ED], jnp.zeros((NT * 128 - ED, 128), W1.dtype)])
    return _mlp(
        x3, x_num, w1p.reshape(NT, 128, 128), W1[ED:],
        b1.reshape(1, 128), g1.reshape(1, 128), be1.reshape(1, 128),
        W2, b2.reshape(1, 64), g2.reshape(1, 64), be2.reshape(1, 64),
        W3, b3.reshape(1, 1),
    )
